# baseline (speedup 1.0000x reference)
import jax
import jax.numpy as jnp
from jax import lax
from jax.experimental import pallas as pl
from jax.experimental.pallas import tpu as pltpu


def _moe_dense_kernel(e_ref, x_ref, w0_ref, b0_ref, w1_ref, b1_ref, o_ref,
                      w0c_ref, w1c_ref, *, tm):
    m = pl.program_id(1)

    @pl.when(m == 0)
    def _cast_weights():
        w0c_ref[...] = w0_ref[...].astype(jnp.bfloat16)
        w1c_ref[...] = w1_ref[...].astype(jnp.bfloat16)

    xb = x_ref[pl.ds(m * tm, tm), :]
    h0 = lax.dot_general(xb, w0c_ref[...], (((1,), (1,)), ((), ())),
                         preferred_element_type=jnp.float32)
    h1 = lax.dot_general(xb, w1c_ref[...], (((1,), (1,)), ((), ())),
                         preferred_element_type=jnp.float32)
    h0 = h0 + b0_ref[0, 0, :][None, :]
    h1 = h1 + b1_ref[0, 0, :][None, :]
    e_col = e_ref[0, 0, :]
    o_ref[...] = jnp.where(e_col[:, None] == 0, h0, h1)


def kernel(x, Wg, bg, W0, b0, W1, b1):
    Bb, Nn, C = x.shape
    T = Bb * Nn
    H = W0.shape[0]
    inp = x.reshape(T, C)

    logits = inp @ Wg.T + bg
    _, top_idx = lax.top_k(logits, 1)
    e = top_idx[:, 0].astype(jnp.int32)

    inp16 = inp.astype(jnp.bfloat16)

    TM = min(512, T)
    TH = min(1024, H)
    m_tiles = T // TM
    h_tiles = H // TH

    e3 = e.reshape(m_tiles, 1, TM)
    b0r = b0.reshape(h_tiles, 1, TH)
    b1r = b1.reshape(h_tiles, 1, TH)

    import functools
    out = pl.pallas_call(
        functools.partial(_moe_dense_kernel, tm=TM),
        grid=(h_tiles, m_tiles),
        in_specs=[
            pl.BlockSpec((1, 1, TM), lambda h, m: (m, 0, 0)),
            pl.BlockSpec((T, C), lambda h, m: (0, 0)),
            pl.BlockSpec((TH, C), lambda h, m: (h, 0)),
            pl.BlockSpec((1, 1, TH), lambda h, m: (h, 0, 0)),
            pl.BlockSpec((TH, C), lambda h, m: (h, 0)),
            pl.BlockSpec((1, 1, TH), lambda h, m: (h, 0, 0)),
        ],
        out_specs=pl.BlockSpec((TM, TH), lambda h, m: (m, h)),
        out_shape=jax.ShapeDtypeStruct((T, H), jnp.float32),
        scratch_shapes=[
            pltpu.VMEM((TH, C), jnp.bfloat16),
            pltpu.VMEM((TH, C), jnp.bfloat16),
        ],
        compiler_params=pltpu.CompilerParams(
            dimension_semantics=("arbitrary", "arbitrary"),
            vmem_limit_bytes=100 * 1024 * 1024,
        ),
    )(e3, inp16, W0, b0r, W1, b1r)
    return out.reshape(Bb, Nn, H)


# speedup vs baseline: 1.1200x; 1.1200x over previous
import functools

import jax
import jax.numpy as jnp
from jax import lax
from jax.experimental import pallas as pl
from jax.experimental.pallas import tpu as pltpu


def _moe_dense_kernel(e_ref, x_ref, w0_ref, b0_ref, w1_ref, b1_ref, o_ref):
    xb = x_ref[...]
    w0b = w0_ref[...].astype(jnp.bfloat16)
    w1b = w1_ref[...].astype(jnp.bfloat16)
    h0 = lax.dot_general(xb, w0b, (((1,), (1,)), ((), ())),
                         preferred_element_type=jnp.float32)
    h1 = lax.dot_general(xb, w1b, (((1,), (1,)), ((), ())),
                         preferred_element_type=jnp.float32)
    h0 = h0 + b0_ref[0, 0, :][None, :]
    h1 = h1 + b1_ref[0, 0, :][None, :]
    e_col = e_ref[0, 0, :]
    o_ref[...] = jnp.where(e_col[:, None] == 0, h0, h1)


def kernel(x, Wg, bg, W0, b0, W1, b1):
    Bb, Nn, C = x.shape
    T = Bb * Nn
    H = W0.shape[0]
    inp = x.reshape(T, C)

    logits = inp @ Wg.T + bg
    _, top_idx = lax.top_k(logits, 1)
    e = top_idx[:, 0].astype(jnp.int32)

    inp16 = inp.astype(jnp.bfloat16)

    TH = min(256, H)
    h_tiles = H // TH

    e3 = e.reshape(1, 1, T)
    b0r = b0.reshape(h_tiles, 1, TH)
    b1r = b1.reshape(h_tiles, 1, TH)

    out = pl.pallas_call(
        _moe_dense_kernel,
        grid=(h_tiles,),
        in_specs=[
            pl.BlockSpec((1, 1, T), lambda h: (0, 0, 0)),
            pl.BlockSpec((T, C), lambda h: (0, 0)),
            pl.BlockSpec((TH, C), lambda h: (h, 0)),
            pl.BlockSpec((1, 1, TH), lambda h: (h, 0, 0)),
            pl.BlockSpec((TH, C), lambda h: (h, 0)),
            pl.BlockSpec((1, 1, TH), lambda h: (h, 0, 0)),
        ],
        out_specs=pl.BlockSpec((T, TH), lambda h: (0, h)),
        out_shape=jax.ShapeDtypeStruct((T, H), jnp.float32),
        compiler_params=pltpu.CompilerParams(
            dimension_semantics=("arbitrary",),
            vmem_limit_bytes=100 * 1024 * 1024,
        ),
    )(e3, inp16, W0, b0r, W1, b1r)
    return out.reshape(Bb, Nn, H)


# speedup vs baseline: 1.1768x; 1.0508x over previous
import functools

import jax
import jax.numpy as jnp
from jax import lax
from jax.experimental import pallas as pl
from jax.experimental.pallas import tpu as pltpu


def _moe_dense_kernel(e_ref, x_ref, w0_ref, b0_ref, w1_ref, b1_ref, o_ref):
    xb = x_ref[...]
    w0b = w0_ref[...].astype(jnp.bfloat16)
    w1b = w1_ref[...].astype(jnp.bfloat16)
    h0 = lax.dot_general(xb, w0b, (((1,), (1,)), ((), ())),
                         preferred_element_type=jnp.float32)
    h1 = lax.dot_general(xb, w1b, (((1,), (1,)), ((), ())),
                         preferred_element_type=jnp.float32)
    h0 = h0 + b0_ref[0, 0, :][None, :]
    h1 = h1 + b1_ref[0, 0, :][None, :]
    e_col = e_ref[0, 0, :]
    o_ref[...] = jnp.where(e_col[:, None] == 0, h0, h1)


def kernel(x, Wg, bg, W0, b0, W1, b1):
    Bb, Nn, C = x.shape
    T = Bb * Nn
    H = W0.shape[0]
    inp = x.reshape(T, C)

    logits = inp @ Wg.T + bg
    e = jnp.argmax(logits, axis=1).astype(jnp.int32)

    inp16 = inp.astype(jnp.bfloat16)

    TH = min(256, H)
    h_tiles = H // TH

    e3 = e.reshape(1, 1, T)
    b0r = b0.reshape(h_tiles, 1, TH)
    b1r = b1.reshape(h_tiles, 1, TH)

    out = pl.pallas_call(
        _moe_dense_kernel,
        grid=(h_tiles,),
        in_specs=[
            pl.BlockSpec((1, 1, T), lambda h: (0, 0, 0)),
            pl.BlockSpec((T, C), lambda h: (0, 0)),
            pl.BlockSpec((TH, C), lambda h: (h, 0)),
            pl.BlockSpec((1, 1, TH), lambda h: (h, 0, 0)),
            pl.BlockSpec((TH, C), lambda h: (h, 0)),
            pl.BlockSpec((1, 1, TH), lambda h: (h, 0, 0)),
        ],
        out_specs=pl.BlockSpec((T, TH), lambda h: (0, h)),
        out_shape=jax.ShapeDtypeStruct((T, H), jnp.float32),
        compiler_params=pltpu.CompilerParams(
            dimension_semantics=("arbitrary",),
            vmem_limit_bytes=100 * 1024 * 1024,
        ),
    )(e3, inp16, W0, b0r, W1, b1r)
    return out.reshape(Bb, Nn, H)
